# Initial kernel scaffold; baseline (speedup 1.0000x reference)
import jax
import jax.numpy as jnp
from jax.experimental import pallas as pl


def kernel(inputs, mask_inds):
    raise NotImplementedError("write your pallas kernel here")


# speedup vs baseline: 21.6362x; 21.6362x over previous
import functools

import jax
import jax.numpy as jnp
from jax import lax
from jax.experimental import pallas as pl
from jax.experimental.pallas import tpu as pltpu
from jax.experimental.pallas import tpu_sc as plsc

B, N, M = 16384, 1000, 200
SCALE = float(N) / float(N - M)

NC, NS, L = 2, 16, 16
NW = NC * NS
ROWS_PER_W = B // NW
RC = 32
NCHUNK = ROWS_PER_W // RC
CH = RC * N
ICH = RC * M
MFULL = M // L
MTAIL = M % L


def _sc_dropout(inputs_flat, inds_flat):
    mesh = plsc.VectorSubcoreMesh(core_axis_name="c", subcore_axis_name="s")

    @functools.partial(
        pl.kernel,
        mesh=mesh,
        compiler_params=pltpu.CompilerParams(needs_layout_passes=False),
        out_type=jax.ShapeDtypeStruct((B * N,), jnp.float32),
        scratch_types=[
            pltpu.VMEM((CH,), jnp.float32),
            pltpu.VMEM((ICH + L,), jnp.int32),
        ],
    )
    def k(in_hbm, idx_hbm, out_hbm, buf, idx_buf):
        wid = lax.axis_index("s") * NC + lax.axis_index("c")
        tail_mask = lax.iota(jnp.int32, L) < MTAIL
        zeros = jnp.zeros((L,), jnp.float32)

        def chunk_body(c, carry):
            ebase = pl.multiple_of(wid * (ROWS_PER_W * N) + c * CH, CH)
            ibase = pl.multiple_of(wid * (ROWS_PER_W * M) + c * ICH, ICH)
            pltpu.sync_copy(in_hbm.at[pl.ds(ebase, CH)], buf)
            pltpu.sync_copy(idx_hbm.at[pl.ds(ibase, ICH)],
                            idx_buf.at[pl.ds(0, ICH)])

            def mul_body(i, carry2):
                base = i * (L * 8)
                for u in range(8):
                    sl = pl.ds(base + u * L, L)
                    buf[sl] = buf[sl] * SCALE
                return carry2

            lax.fori_loop(0, CH // (L * 8), mul_body, 0, unroll=1)

            def row_body(r, carry2):
                rb = r * M
                off = r * N
                for j in range(MFULL):
                    iv = idx_buf[pl.ds(rb + j * L, L)] + off
                    plsc.store_scatter(buf, [iv], zeros)
                iv = idx_buf[pl.ds(rb + MFULL * L, L)] + off
                plsc.store_scatter(buf, [iv], zeros, mask=tail_mask)
                return carry2

            lax.fori_loop(0, RC, row_body, 0, unroll=1)

            pltpu.sync_copy(buf, out_hbm.at[pl.ds(ebase, CH)])
            return carry

        lax.fori_loop(0, NCHUNK, chunk_body, 0, unroll=1)

    return k(inputs_flat, inds_flat)


@jax.jit
def kernel(inputs, mask_inds):
    out = _sc_dropout(inputs.reshape(-1), mask_inds.reshape(-1))
    return out.reshape(B, N)


# speedup vs baseline: 25.3526x; 1.1718x over previous
import functools

import jax
import jax.numpy as jnp
from jax import lax
from jax.experimental import pallas as pl
from jax.experimental.pallas import tpu as pltpu
from jax.experimental.pallas import tpu_sc as plsc

B, N, M = 16384, 1000, 200
SCALE = float(N) / float(N - M)

NC, NS, L = 2, 16, 16
NW = NC * NS
ROWS_PER_W = B // NW
RC = 32
NCHUNK = ROWS_PER_W // RC
CH = RC * N
ICH = RC * M
MFULL = M // L
MTAIL = M % L
NBUF = 3


def _sc_dropout(inputs_flat, inds_flat):
    mesh = plsc.VectorSubcoreMesh(core_axis_name="c", subcore_axis_name="s")

    @functools.partial(
        pl.kernel,
        mesh=mesh,
        compiler_params=pltpu.CompilerParams(needs_layout_passes=False),
        out_type=jax.ShapeDtypeStruct((B * N,), jnp.float32),
        scratch_types=(
            [pltpu.VMEM((CH,), jnp.float32) for _ in range(NBUF)]
            + [pltpu.VMEM((ICH + L,), jnp.int32) for _ in range(NBUF)]
            + [pltpu.SemaphoreType.DMA for _ in range(3 * NBUF)]
        ),
    )
    def k(in_hbm, idx_hbm, out_hbm, *scratch):
        bufs = scratch[0:NBUF]
        idx_bufs = scratch[NBUF:2 * NBUF]
        in_sems = scratch[2 * NBUF:3 * NBUF]
        idx_sems = scratch[3 * NBUF:4 * NBUF]
        out_sems = scratch[4 * NBUF:5 * NBUF]

        wid = lax.axis_index("s") * NC + lax.axis_index("c")
        tail_mask = lax.iota(jnp.int32, L) < MTAIL
        zeros = jnp.zeros((L,), jnp.float32)
        row0 = wid * ROWS_PER_W

        def load_descs(c, b):
            ebase = pl.multiple_of((row0 + c * RC) * N, CH)
            ibase = pl.multiple_of((row0 + c * RC) * M, ICH)
            return (
                pltpu.make_async_copy(
                    in_hbm.at[pl.ds(ebase, CH)], bufs[b], in_sems[b]),
                pltpu.make_async_copy(
                    idx_hbm.at[pl.ds(ibase, ICH)],
                    idx_bufs[b].at[pl.ds(0, ICH)], idx_sems[b]),
            )

        def store_desc(c, b):
            ebase = pl.multiple_of((row0 + c * RC) * N, CH)
            return pltpu.make_async_copy(
                bufs[b], out_hbm.at[pl.ds(ebase, CH)], out_sems[b])

        def compute(b):
            buf, idx_buf = bufs[b], idx_bufs[b]

            def mul_body(i, carry):
                base = i * (L * 8)
                for u in range(8):
                    sl = pl.ds(base + u * L, L)
                    buf[sl] = buf[sl] * SCALE
                return carry

            lax.fori_loop(0, CH // (L * 8), mul_body, 0, unroll=1)

            def row_body(r, carry):
                rb = r * M
                off = r * N
                for j in range(MFULL):
                    iv = idx_buf[pl.ds(rb + j * L, L)] + off
                    plsc.store_scatter(buf, [iv], zeros)
                iv = idx_buf[pl.ds(rb + MFULL * L, L)] + off
                plsc.store_scatter(buf, [iv], zeros, mask=tail_mask)
                return carry

            lax.fori_loop(0, RC, row_body, 0, unroll=1)

        for d in load_descs(0, 0):
            d.start()
        for c in range(NCHUNK):
            b = c % NBUF
            if c + 1 < NCHUNK:
                bn = (c + 1) % NBUF
                if c + 1 >= NBUF:
                    store_desc(c + 1 - NBUF, bn).wait()
                for d in load_descs(c + 1, bn):
                    d.start()
            for d in load_descs(c, b):
                d.wait()
            compute(b)
            store_desc(c, b).start()
        for c in range(NCHUNK - NBUF, NCHUNK):
            store_desc(c, c % NBUF).wait()

    return k(inputs_flat, inds_flat)


@jax.jit
def kernel(inputs, mask_inds):
    out = _sc_dropout(inputs.reshape(-1), mask_inds.reshape(-1))
    return out.reshape(B, N)


# speedup vs baseline: 38.3206x; 1.5115x over previous
import functools

import jax
import jax.numpy as jnp
from jax import lax
from jax.experimental import pallas as pl
from jax.experimental.pallas import tpu as pltpu
from jax.experimental.pallas import tpu_sc as plsc

B, N, M = 16384, 1000, 200
SCALE = float(N) / float(N - M)

NC, NS, L = 2, 16, 16
NW = NC * NS
ROWS_PER_W = B // NW
RC = 32
NCHUNK = ROWS_PER_W // RC
NFULL = N // L
MFULL = M // L
MTAIL = M % L
NBUF = 3


def _sc_dropout(inputs, mask_inds):
    mesh = plsc.VectorSubcoreMesh(core_axis_name="c", subcore_axis_name="s")

    @functools.partial(
        pl.kernel,
        mesh=mesh,
        compiler_params=pltpu.CompilerParams(needs_layout_passes=False),
        out_type=jax.ShapeDtypeStruct((B, N), jnp.float32),
        scratch_types=(
            [pltpu.VMEM((RC, N), jnp.float32) for _ in range(NBUF)]
            + [pltpu.VMEM((RC, M), jnp.int32) for _ in range(NBUF)]
            + [pltpu.SemaphoreType.DMA for _ in range(3 * NBUF)]
        ),
    )
    def k(in_hbm, idx_hbm, out_hbm, *scratch):
        bufs = scratch[0:NBUF]
        idx_bufs = scratch[NBUF:2 * NBUF]
        in_sems = scratch[2 * NBUF:3 * NBUF]
        idx_sems = scratch[3 * NBUF:4 * NBUF]
        out_sems = scratch[4 * NBUF:5 * NBUF]

        wid = lax.axis_index("s") * NC + lax.axis_index("c")
        lanes = lax.iota(jnp.int32, L)
        hi_lanes = lanes >= (L - (N - NFULL * L))
        zeros = jnp.zeros((L,), jnp.float32)
        zeros_i = jnp.zeros((L,), jnp.int32)
        row0 = wid * ROWS_PER_W

        def load_descs(c, b):
            r0 = pl.multiple_of(row0 + c * RC, RC)
            return (
                pltpu.make_async_copy(
                    in_hbm.at[pl.ds(r0, RC)], bufs[b], in_sems[b]),
                pltpu.make_async_copy(
                    idx_hbm.at[pl.ds(r0, RC)], idx_bufs[b], idx_sems[b]),
            )

        def store_desc(c, b):
            r0 = pl.multiple_of(row0 + c * RC, RC)
            return pltpu.make_async_copy(
                bufs[b], out_hbm.at[pl.ds(r0, RC)], out_sems[b])

        def compute(b):
            buf, idx_buf = bufs[b], idx_bufs[b]

            def row_body(r, carry):
                for j in range(NFULL):
                    sl = pl.ds(j * L, L)
                    buf[r, sl] = buf[r, sl] * SCALE
                tl = pl.ds(N - L, L)
                t = buf[r, tl]
                buf[r, tl] = jnp.where(hi_lanes, t * SCALE, t)
                rv = zeros_i + r
                for j in range(MFULL):
                    iv = idx_buf[r, pl.ds(j * L, L)]
                    plsc.store_scatter(buf, [rv, iv], zeros)
                iv = idx_buf[r, pl.ds(M - L, L)]
                plsc.store_scatter(buf, [rv, iv], zeros)
                return carry

            lax.fori_loop(0, RC, row_body, 0, unroll=1)

        for d in load_descs(0, 0):
            d.start()
        for c in range(NCHUNK):
            b = c % NBUF
            if c + 1 < NCHUNK:
                bn = (c + 1) % NBUF
                if c + 1 >= NBUF:
                    store_desc(c + 1 - NBUF, bn).wait()
                for d in load_descs(c + 1, bn):
                    d.start()
            for d in load_descs(c, b):
                d.wait()
            compute(b)
            store_desc(c, b).start()
        for c in range(NCHUNK - NBUF, NCHUNK):
            store_desc(c, c % NBUF).wait()

    return k(inputs, mask_inds)


@jax.jit
def kernel(inputs, mask_inds):
    return _sc_dropout(inputs, mask_inds)
